# Initial kernel scaffold; baseline (speedup 1.0000x reference)
import jax
import jax.numpy as jnp
from jax.experimental import pallas as pl


def kernel(inputs):
    raise NotImplementedError("write your pallas kernel here")


# speedup vs baseline: 1.4413x; 1.4413x over previous
import functools

import jax
import jax.numpy as jnp
import numpy as np
from jax import lax
from jax.experimental import pallas as pl
from jax.experimental.pallas import tpu as pltpu
from jax.experimental.pallas import tpu_sc as plsc

_PROB = 0.5
_NUM_PREFIX = 1
_B, _T, _D = 128, 577, 768
_NP = _T - _NUM_PREFIX
_NK = max(1, int(_NP * (1.0 - _PROB)))
_ROWS = _NUM_PREFIX + _NK
_NW = 32
_BPW = _B // _NW
_CH = 64
_NCH = 4
_TAIL = _ROWS - _NCH * _CH


@functools.lru_cache(maxsize=1)
def _row_indices() -> np.ndarray:
    with jax.ensure_compile_time_eval():
        rand = jax.random.normal(jax.random.key(42), (_B, _NP), dtype=jnp.float32)
        order = jnp.argsort(rand, axis=-1)
        keep = jnp.sort(order[:, :_NK], axis=-1) + _NUM_PREFIX
        full = jnp.concatenate(
            [jnp.zeros((_B, _NUM_PREFIX), keep.dtype), keep], axis=1)
    return np.asarray(full).astype(np.int32)


def _sc_gather(x, src_main, src_tail, dst_tail):
    mesh = plsc.VectorSubcoreMesh(core_axis_name="c", subcore_axis_name="s")

    @functools.partial(
        pl.kernel,
        mesh=mesh,
        out_type=jax.ShapeDtypeStruct((_B, _ROWS, _D), jnp.float32),
        scratch_types=[
            pltpu.VMEM((_NCH, _CH), jnp.int32),
            pltpu.VMEM((_TAIL,), jnp.int32),
            pltpu.VMEM((_CH, _D), jnp.float32),
            pltpu.VMEM((_TAIL, _D), jnp.float32),
            pltpu.SemaphoreType.DMA,
        ],
    )
    def gather_rows(x_hbm, srcm_hbm, srct_hbm, out_hbm,
                    idxm_v, idxt_v, buf, tbuf, gsem):
        wid = lax.axis_index("s") * 2 + lax.axis_index("c")
        for i in range(_BPW):
            b = wid * _BPW + i
            pltpu.sync_copy(srcm_hbm.at[b], idxm_v)
            pltpu.sync_copy(srct_hbm.at[b], idxt_v)
            for g in range(_NCH):
                pltpu.async_copy(x_hbm.at[b].at[idxm_v.at[g]], buf, gsem).wait()
                pltpu.sync_copy(buf, out_hbm.at[b, pl.ds(g * _CH, _CH)])
            pltpu.async_copy(x_hbm.at[b].at[idxt_v], tbuf, gsem).wait()
            pltpu.sync_copy(tbuf, out_hbm.at[b, pl.ds(_NCH * _CH, _TAIL)])

    return gather_rows(x, src_main, src_tail)


def kernel(inputs):
    x = inputs
    rows = _row_indices()
    src_main = jnp.asarray(rows[:, : _NCH * _CH].reshape(_B, _NCH, _CH))
    src_tail = jnp.asarray(rows[:, _NCH * _CH :])
    dst_tail = jnp.asarray(
        np.arange(_NCH * _CH, _ROWS, dtype=np.int32).reshape(1, _TAIL))
    return _sc_gather(x, src_main, src_tail, dst_tail)


# speedup vs baseline: 1.5009x; 1.0414x over previous
import functools

import jax
import jax.numpy as jnp
import numpy as np
from jax import lax
from jax.experimental import pallas as pl
from jax.experimental.pallas import tpu as pltpu
from jax.experimental.pallas import tpu_sc as plsc

_PROB = 0.5
_NUM_PREFIX = 1
_B, _T, _D = 128, 577, 768
_NP = _T - _NUM_PREFIX
_NK = max(1, int(_NP * (1.0 - _PROB)))
_ROWS = _NUM_PREFIX + _NK
_NW = 32
_BPW = _B // _NW
_CH = 72
_NCH = 4
_TAIL = _ROWS - _NCH * _CH


@functools.lru_cache(maxsize=1)
def _row_indices() -> np.ndarray:
    with jax.ensure_compile_time_eval():
        rand = jax.random.normal(jax.random.key(42), (_B, _NP), dtype=jnp.float32)
        order = jnp.argsort(rand, axis=-1)
        keep = jnp.sort(order[:, :_NK], axis=-1) + _NUM_PREFIX
        full = jnp.concatenate(
            [jnp.zeros((_B, _NUM_PREFIX), keep.dtype), keep], axis=1)
    return np.asarray(full).astype(np.int32)


def _sc_gather(x, src_main, src_tail, dst_tail):
    mesh = plsc.VectorSubcoreMesh(core_axis_name="c", subcore_axis_name="s")

    @functools.partial(
        pl.kernel,
        mesh=mesh,
        out_type=jax.ShapeDtypeStruct((_B, _ROWS, _D), jnp.float32),
        scratch_types=[
            pltpu.VMEM((_BPW, _NCH, _CH), jnp.int32),
            pltpu.VMEM((_BPW, 1, _TAIL), jnp.int32),
            pltpu.VMEM((1, _TAIL), jnp.int32),
            pltpu.VMEM((_CH, _D), jnp.float32),
            pltpu.VMEM((_CH, _D), jnp.float32),
            pltpu.VMEM((_TAIL, _D), jnp.float32),
            pltpu.SemaphoreType.DMA,
            pltpu.SemaphoreType.DMA,
            pltpu.SemaphoreType.DMA,
            pltpu.SemaphoreType.DMA,
            pltpu.SemaphoreType.DMA,
            pltpu.SemaphoreType.DMA,
        ],
    )
    def gather_rows(x_hbm, srcm_hbm, srct_hbm, dstt_hbm, out_hbm,
                    idxm_v, idxt_v, widx_v, buf0, buf1, tbuf,
                    gsem0, gsem1, gsemt, wsem0, wsem1, wsemt):
        wid = lax.axis_index("s") * 2 + lax.axis_index("c")
        base = wid * _BPW
        pltpu.sync_copy(srcm_hbm.at[pl.ds(base, _BPW)], idxm_v)
        pltpu.sync_copy(srct_hbm.at[pl.ds(base, _BPW)], idxt_v)
        pltpu.sync_copy(dstt_hbm, widx_v)

        bufs = {0: buf0, 1: buf1, "t": tbuf}
        gsems = {0: gsem0, 1: gsem1, "t": gsemt}
        wsems = {0: wsem0, 1: wsem1, "t": wsemt}

        jobs = []
        for i in range(_BPW):
            for g in range(_NCH):
                jobs.append(("m", i, g, (i * _NCH + g) % 2))
            jobs.append(("t", i, 0, "t"))

        gds = [None] * len(jobs)
        last_write = {0: None, 1: None, "t": None}

        def idx_ref(job):
            kind, i, g, _ = job
            if kind == "m":
                return idxm_v.at[i].at[g]
            return idxt_v.at[i].at[0]

        def start_gather(j):
            kind, i, g, tag = jobs[j]
            if last_write[tag] is not None:
                last_write[tag].wait()
                last_write[tag] = None
            gds[j] = pltpu.async_copy(
                x_hbm.at[base + i].at[idx_ref(jobs[j])], bufs[tag], gsems[tag])

        start_gather(0)
        for j in range(len(jobs)):
            if j + 1 < len(jobs):
                start_gather(j + 1)
            gds[j].wait()
            kind, i, g, tag = jobs[j]
            if kind == "m":
                last_write[tag] = pltpu.async_copy(
                    bufs[tag], out_hbm.at[base + i, pl.ds(g * _CH, _CH)],
                    wsems[tag])
            else:
                last_write[tag] = pltpu.async_copy(
                    tbuf, out_hbm.at[base + i].at[widx_v.at[0]], wsems[tag])
        for tag in (0, 1, "t"):
            if last_write[tag] is not None:
                last_write[tag].wait()

    return gather_rows(x, src_main, src_tail, dst_tail)


def kernel(inputs):
    x = inputs
    rows = _row_indices()
    src_main = jnp.asarray(rows[:, : _NCH * _CH].reshape(_B, _NCH, _CH))
    src_tail = jnp.asarray(rows[:, _NCH * _CH :].reshape(_B, 1, _TAIL))
    dst_tail = jnp.asarray(
        np.arange(_NCH * _CH, _ROWS, dtype=np.int32).reshape(1, _TAIL))
    return _sc_gather(x, src_main, src_tail, dst_tail)
